# baseline (speedup 1.0000x reference)
import functools

import jax
import jax.numpy as jnp
from jax import lax
from jax.experimental import pallas as pl
from jax.experimental.pallas import tpu as pltpu
from jax.experimental.pallas import tpu_sc as plsc

N = 10000
E = 320000
D = 128
H = 128
C = 64

NC = 2
NS = 16
NW = NC * NS

CH = 128
EW = 10240
EP = NW * EW
G = EW // CH
GC0 = 104
GC1 = 56
NP = 10112
RPT = NP // NS


def _stripe_copy(src_get, dst_get):
    nfull = RPT // CH
    tail = RPT % CH
    for k in range(nfull):
        off = k * CH
        _src = src_get(off, CH)
        _dst = dst_get(off, CH)
        pltpu.sync_copy(_src, _dst)
    pltpu.sync_copy(src_get(nfull * CH, tail), dst_get(nfull * CH, tail))


def _agg_body(table, sd2, psum, acc, *refs, W):
    srcb = refs[0:4]
    dstb = refs[4:8]
    rows = refs[8]
    semi = refs[9:13]
    semg = refs[13:15]
    c = lax.axis_index("c")
    s = lax.axis_index("s")
    cbase = jnp.where(c == 0, s * GC0, NS * GC0 + s * GC1)
    gc = jnp.where(c == 0, GC0, GC1)
    base = s * RPT

    def _zrow(i, carry):
        for j in range(W // 16):
            rows[0, i, pl.ds(j * 16, 16)] = jnp.zeros((16,), jnp.float32)
        return carry

    lax.fori_loop(0, CH, _zrow, 0)

    _stripe_copy(lambda o, n: rows.at[0, pl.ds(0, n)],
                 lambda o, n: acc.at[pl.ds(base + o, n)])
    plsc.subcore_barrier()

    def _issue_idx(g, i):
        pltpu.async_copy(sd2.at[cbase + g, 0], srcb[i], semi[i])
        pltpu.async_copy(sd2.at[cbase + g, 1], dstb[i], semi[i])

    def _wait_idx(g, i):
        pltpu.make_async_copy(sd2.at[cbase + g, 0], srcb[i], semi[i]).wait()
        pltpu.make_async_copy(sd2.at[cbase + g, 1], dstb[i], semi[i]).wait()

    def _issue_gather(i, b):
        pltpu.async_copy(table.at[srcb[i]], rows.at[b], semg[b])

    def _wait_gather(i, b):
        pltpu.make_async_copy(table.at[srcb[i]], rows.at[b], semg[b]).wait()

    def _scat(i, b):
        pltpu.sync_copy(rows.at[b], acc.at[dstb[i]], add=True)

    for g in range(4):
        _issue_idx(g, g)
    _wait_idx(0, 0)
    _issue_gather(0, 0)

    def _body(gg, carry):
        for k in range(4):
            cc = gg * 4 + k
            r = k % 2
            r1 = (k + 1) % 2
            i1 = (k + 1) % 4

            @pl.when(cc + 1 < gc)
            def _():
                _wait_idx(cc + 1, i1)
                _issue_gather(i1, r1)

            _wait_gather(k, r)
            _scat(k, r)

            @pl.when(cc + 4 < gc)
            def _():
                _issue_idx(cc + 4, k)
        return carry

    lax.fori_loop(0, gc // 4, _body, 0)
    plsc.subcore_barrier()

    _stripe_copy(lambda o, n: acc.at[pl.ds(base + o, n)],
                 lambda o, n: psum.at[c, pl.ds(base + o, n)])


def _cnt_body(sd2, pcnt, acc, db0, db1, db2, db3, rows, si0, si1, si2, si3):
    dstb = (db0, db1, db2, db3)
    semi = (si0, si1, si2, si3)
    c = lax.axis_index("c")
    s = lax.axis_index("s")
    cbase = jnp.where(c == 0, s * GC0, NS * GC0 + s * GC1)
    gc = jnp.where(c == 0, GC0, GC1)
    base = s * RPT

    def _zrow(i, carry):
        for j in range(H // 16):
            rows[i, pl.ds(j * 16, 16)] = jnp.zeros((16,), jnp.float32)
        return carry

    lax.fori_loop(0, CH, _zrow, 0)
    _stripe_copy(lambda o, n: rows.at[pl.ds(0, n)],
                 lambda o, n: acc.at[pl.ds(base + o, n)])

    def _orow(i, carry):
        for j in range(H // 16):
            rows[i, pl.ds(j * 16, 16)] = jnp.ones((16,), jnp.float32)
        return carry

    lax.fori_loop(0, CH, _orow, 0)
    plsc.subcore_barrier()

    def _issue_idx(g, i):
        pltpu.async_copy(sd2.at[cbase + g, 1], dstb[i], semi[i])

    def _wait_idx(g, i):
        pltpu.make_async_copy(sd2.at[cbase + g, 1], dstb[i], semi[i]).wait()

    for g in range(4):
        _issue_idx(g, g)

    def _body(gg, carry):
        for k in range(4):
            cc = gg * 4 + k
            _wait_idx(cc, k)
            pltpu.sync_copy(rows, acc.at[dstb[k]], add=True)

            @pl.when(cc + 4 < gc)
            def _():
                _issue_idx(cc + 4, k)
        return carry

    lax.fori_loop(0, gc // 4, _body, 0)
    plsc.subcore_barrier()
    _stripe_copy(lambda o, n: acc.at[pl.ds(base + o, n)],
                 lambda o, n: pcnt.at[c, pl.ds(base + o, n)])


def _sc_mesh():
    return plsc.VectorSubcoreMesh(core_axis_name="c", subcore_axis_name="s",
                                  num_cores=NC, num_subcores=NS)


def _make_agg(W):
    idx = [pltpu.VMEM((CH,), jnp.int32)] * 8
    sems = [pltpu.SemaphoreType.DMA] * 6
    return pl.kernel(
        functools.partial(_agg_body, W=W),
        out_type=jax.ShapeDtypeStruct((NC, NP, W), jnp.float32),
        mesh=_sc_mesh(),
        scratch_types=[pltpu.VMEM_SHARED((NP, W), jnp.float32)] + idx
        + [pltpu.VMEM((2, CH, W), jnp.float32)] + sems,
        name=f"sage_sc_agg{W}",
    )


def _make_cnt():
    return pl.kernel(
        _cnt_body,
        out_type=jax.ShapeDtypeStruct((NC, NP, H), jnp.float32),
        mesh=_sc_mesh(),
        scratch_types=[pltpu.VMEM_SHARED((NP, H), jnp.float32)]
        + [pltpu.VMEM((CH,), jnp.int32)] * 4
        + [pltpu.VMEM((CH, H), jnp.float32)]
        + [pltpu.SemaphoreType.DMA] * 4,
        name="sage_sc_cnt",
    )


RB = 400
GRID = N // RB


def _inv_cnt(c_r):
    cnt = c_r[0, :, 0:1] + c_r[1, :, 0:1]
    return 1.0 / jnp.maximum(cnt, 1.0)


def _tc_body(x_r, p_r, c_r, ws_r, wn_r, b_r, o_r, *, relu):
    neigh = (p_r[0] + p_r[1]) * _inv_cnt(c_r)
    acc = jnp.dot(x_r[...], ws_r[...], preferred_element_type=jnp.float32)
    acc = acc + jnp.dot(neigh, wn_r[...], preferred_element_type=jnp.float32)
    acc = acc + b_r[...]
    o_r[...] = jnp.maximum(acc, 0.0) if relu else acc


def _node_spec(w):
    return pl.BlockSpec((RB, w), lambda i: (i, 0))


def _part_spec(w):
    return pl.BlockSpec((NC, RB, w), lambda i: (0, i, 0))


def _full_spec(a, b):
    return pl.BlockSpec((a, b), lambda i: (0, 0))


def _make_tc(din, dout, relu, name):
    return pl.pallas_call(
        functools.partial(_tc_body, relu=relu),
        grid=(GRID,),
        in_specs=[_node_spec(din), _part_spec(din), _part_spec(din),
                  _full_spec(din, dout), _full_spec(din, dout),
                  _full_spec(1, dout)],
        out_specs=_node_spec(dout),
        out_shape=jax.ShapeDtypeStruct((N, dout), jnp.float32),
        name=name,
    )


_tc1 = _make_tc(D, H, True, "sage_tc1")
_tc2 = _make_tc(H, H, True, "sage_tc2")
_tc3 = _make_tc(H, C, False, "sage_tc3")


def kernel(feats, edge_index, Ws1, Wn1, b1, Ws2, Wn2, b2, Ws3, Wn3, b3):
    pad = EP - E
    src = jnp.concatenate([edge_index[0], jnp.zeros((pad,), jnp.int32)])
    dst = jnp.concatenate([edge_index[1], jnp.full((pad,), N, jnp.int32)])
    sd2 = jnp.stack([src.reshape(NW * G, CH), dst.reshape(NW * G, CH)], axis=1)
    b1r = b1.reshape(1, H)
    b2r = b2.reshape(1, H)
    b3r = b3.reshape(1, C)

    agg128 = _make_agg(H)
    cnt = _make_cnt()(sd2)
    p1 = agg128(feats, sd2)
    h1 = _tc1(feats, p1, cnt, Ws1, Wn1, b1r)
    p2 = agg128(h1, sd2)
    h2 = _tc2(h1, p2, cnt, Ws2, Wn2, b2r)
    p3 = agg128(h2, sd2)
    return _tc3(h2, p3, cnt, Ws3, Wn3, b3r)


# speedup vs baseline: 1.0219x; 1.0219x over previous
import functools

import jax
import jax.numpy as jnp
from jax import lax
from jax.experimental import pallas as pl
from jax.experimental.pallas import tpu as pltpu
from jax.experimental.pallas import tpu_sc as plsc

N = 10000
E = 320000
D = 128
H = 128
C = 64

NC = 2
NS = 16
NW = NC * NS

CH = 128
EW = 10240
EP = NW * EW
G = EW // CH
GC0 = 136
GC1 = 24
NP = 10112
RPT = NP // NS


def _stripe_copy(src_get, dst_get):
    nfull = RPT // CH
    tail = RPT % CH
    for k in range(nfull):
        off = k * CH
        _src = src_get(off, CH)
        _dst = dst_get(off, CH)
        pltpu.sync_copy(_src, _dst)
    pltpu.sync_copy(src_get(nfull * CH, tail), dst_get(nfull * CH, tail))


def _agg_body(table, sd2, psum, acc, *refs, W):
    srcb = refs[0:4]
    dstb = refs[4:8]
    rows = refs[8]
    semi = refs[9:13]
    semg = refs[13:15]
    c = lax.axis_index("c")
    s = lax.axis_index("s")
    cbase = jnp.where(c == 0, s * GC0, NS * GC0 + s * GC1)
    gc = jnp.where(c == 0, GC0, GC1)
    base = s * RPT

    def _zrow(i, carry):
        for j in range(W // 16):
            rows[0, i, pl.ds(j * 16, 16)] = jnp.zeros((16,), jnp.float32)
        return carry

    lax.fori_loop(0, CH, _zrow, 0)

    _stripe_copy(lambda o, n: rows.at[0, pl.ds(0, n)],
                 lambda o, n: acc.at[pl.ds(base + o, n)])
    plsc.subcore_barrier()

    def _issue_idx(g, i):
        pltpu.async_copy(sd2.at[cbase + g, 0], srcb[i], semi[i])
        pltpu.async_copy(sd2.at[cbase + g, 1], dstb[i], semi[i])

    def _wait_idx(g, i):
        pltpu.make_async_copy(sd2.at[cbase + g, 0], srcb[i], semi[i]).wait()
        pltpu.make_async_copy(sd2.at[cbase + g, 1], dstb[i], semi[i]).wait()

    def _issue_gather(i, b):
        pltpu.async_copy(table.at[srcb[i]], rows.at[b], semg[b])

    def _wait_gather(i, b):
        pltpu.make_async_copy(table.at[srcb[i]], rows.at[b], semg[b]).wait()

    def _scat(i, b):
        pltpu.sync_copy(rows.at[b], acc.at[dstb[i]], add=True)

    for g in range(4):
        _issue_idx(g, g)
    _wait_idx(0, 0)
    _issue_gather(0, 0)

    def _body(gg, carry):
        for k in range(4):
            cc = gg * 4 + k
            r = k % 2
            r1 = (k + 1) % 2
            i1 = (k + 1) % 4

            @pl.when(cc + 1 < gc)
            def _():
                _wait_idx(cc + 1, i1)
                _issue_gather(i1, r1)

            _wait_gather(k, r)
            _scat(k, r)

            @pl.when(cc + 4 < gc)
            def _():
                _issue_idx(cc + 4, k)
        return carry

    lax.fori_loop(0, gc // 4, _body, 0)
    plsc.subcore_barrier()

    _stripe_copy(lambda o, n: acc.at[pl.ds(base + o, n)],
                 lambda o, n: psum.at[c, pl.ds(base + o, n)])


def _cnt_body(sd2, pcnt, acc, db0, db1, db2, db3, rows, si0, si1, si2, si3):
    dstb = (db0, db1, db2, db3)
    semi = (si0, si1, si2, si3)
    c = lax.axis_index("c")
    s = lax.axis_index("s")
    cbase = jnp.where(c == 0, s * GC0, NS * GC0 + s * GC1)
    gc = jnp.where(c == 0, GC0, GC1)
    base = s * RPT

    def _zrow(i, carry):
        for j in range(H // 16):
            rows[i, pl.ds(j * 16, 16)] = jnp.zeros((16,), jnp.float32)
        return carry

    lax.fori_loop(0, CH, _zrow, 0)
    _stripe_copy(lambda o, n: rows.at[pl.ds(0, n)],
                 lambda o, n: acc.at[pl.ds(base + o, n)])

    def _orow(i, carry):
        for j in range(H // 16):
            rows[i, pl.ds(j * 16, 16)] = jnp.ones((16,), jnp.float32)
        return carry

    lax.fori_loop(0, CH, _orow, 0)
    plsc.subcore_barrier()

    def _issue_idx(g, i):
        pltpu.async_copy(sd2.at[cbase + g, 1], dstb[i], semi[i])

    def _wait_idx(g, i):
        pltpu.make_async_copy(sd2.at[cbase + g, 1], dstb[i], semi[i]).wait()

    for g in range(4):
        _issue_idx(g, g)

    def _body(gg, carry):
        for k in range(4):
            cc = gg * 4 + k
            _wait_idx(cc, k)
            pltpu.sync_copy(rows, acc.at[dstb[k]], add=True)

            @pl.when(cc + 4 < gc)
            def _():
                _issue_idx(cc + 4, k)
        return carry

    lax.fori_loop(0, gc // 4, _body, 0)
    plsc.subcore_barrier()
    _stripe_copy(lambda o, n: acc.at[pl.ds(base + o, n)],
                 lambda o, n: pcnt.at[c, pl.ds(base + o, n)])


def _sc_mesh():
    return plsc.VectorSubcoreMesh(core_axis_name="c", subcore_axis_name="s",
                                  num_cores=NC, num_subcores=NS)


def _make_agg(W):
    idx = [pltpu.VMEM((CH,), jnp.int32)] * 8
    sems = [pltpu.SemaphoreType.DMA] * 6
    return pl.kernel(
        functools.partial(_agg_body, W=W),
        out_type=jax.ShapeDtypeStruct((NC, NP, W), jnp.float32),
        mesh=_sc_mesh(),
        scratch_types=[pltpu.VMEM_SHARED((NP, W), jnp.float32)] + idx
        + [pltpu.VMEM((2, CH, W), jnp.float32)] + sems,
        name=f"sage_sc_agg{W}",
    )


def _make_cnt():
    return pl.kernel(
        _cnt_body,
        out_type=jax.ShapeDtypeStruct((NC, NP, H), jnp.float32),
        mesh=_sc_mesh(),
        scratch_types=[pltpu.VMEM_SHARED((NP, H), jnp.float32)]
        + [pltpu.VMEM((CH,), jnp.int32)] * 4
        + [pltpu.VMEM((CH, H), jnp.float32)]
        + [pltpu.SemaphoreType.DMA] * 4,
        name="sage_sc_cnt",
    )


RB = 400
GRID = N // RB


def _inv_cnt(c_r):
    cnt = c_r[0, :, 0:1] + c_r[1, :, 0:1]
    return 1.0 / jnp.maximum(cnt, 1.0)


def _tc_body(x_r, p_r, c_r, ws_r, wn_r, b_r, o_r, *, relu):
    neigh = (p_r[0] + p_r[1]) * _inv_cnt(c_r)
    acc = jnp.dot(x_r[...], ws_r[...], preferred_element_type=jnp.float32)
    acc = acc + jnp.dot(neigh, wn_r[...], preferred_element_type=jnp.float32)
    acc = acc + b_r[...]
    o_r[...] = jnp.maximum(acc, 0.0) if relu else acc


def _node_spec(w):
    return pl.BlockSpec((RB, w), lambda i: (i, 0))


def _part_spec(w):
    return pl.BlockSpec((NC, RB, w), lambda i: (0, i, 0))


def _full_spec(a, b):
    return pl.BlockSpec((a, b), lambda i: (0, 0))


def _make_tc(din, dout, relu, name):
    return pl.pallas_call(
        functools.partial(_tc_body, relu=relu),
        grid=(GRID,),
        in_specs=[_node_spec(din), _part_spec(din), _part_spec(din),
                  _full_spec(din, dout), _full_spec(din, dout),
                  _full_spec(1, dout)],
        out_specs=_node_spec(dout),
        out_shape=jax.ShapeDtypeStruct((N, dout), jnp.float32),
        name=name,
    )


_tc1 = _make_tc(D, H, True, "sage_tc1")
_tc2 = _make_tc(H, H, True, "sage_tc2")
_tc3 = _make_tc(H, C, False, "sage_tc3")


def kernel(feats, edge_index, Ws1, Wn1, b1, Ws2, Wn2, b2, Ws3, Wn3, b3):
    pad = EP - E
    src = jnp.concatenate([edge_index[0], jnp.zeros((pad,), jnp.int32)])
    dst = jnp.concatenate([edge_index[1], jnp.full((pad,), N, jnp.int32)])
    sd2 = jnp.stack([src.reshape(NW * G, CH), dst.reshape(NW * G, CH)], axis=1)
    b1r = b1.reshape(1, H)
    b2r = b2.reshape(1, H)
    b3r = b3.reshape(1, C)

    agg128 = _make_agg(H)
    cnt = _make_cnt()(sd2)
    p1 = agg128(feats, sd2)
    h1 = _tc1(feats, p1, cnt, Ws1, Wn1, b1r)
    p2 = agg128(h1, sd2)
    h2 = _tc2(h1, p2, cnt, Ws2, Wn2, b2r)
    p3 = agg128(h2, sd2)
    return _tc3(h2, p3, cnt, Ws3, Wn3, b3r)


# speedup vs baseline: 1.1323x; 1.1080x over previous
import functools

import jax
import jax.numpy as jnp
from jax import lax
from jax.experimental import pallas as pl
from jax.experimental.pallas import tpu as pltpu
from jax.experimental.pallas import tpu_sc as plsc

N = 10000
E = 320000
D = 128
H = 128
C = 64

NC = 2
NS = 16
NW = NC * NS

CH = 128
EW = 10240
EP = NW * EW
G = EW // CH
GC0 = 144
GC1 = 16
NP = 10112
RPT = NP // NS


def _stripe_copy(src_get, dst_get):
    nfull = RPT // CH
    tail = RPT % CH
    for k in range(nfull):
        off = k * CH
        _src = src_get(off, CH)
        _dst = dst_get(off, CH)
        pltpu.sync_copy(_src, _dst)
    pltpu.sync_copy(src_get(nfull * CH, tail), dst_get(nfull * CH, tail))


def _agg_body(table, sd2, psum, acc, *refs, W):
    srcb = refs[0:4]
    dstb = refs[4:8]
    rows = refs[8]
    semi = refs[9:13]
    semg = refs[13:15]
    c = lax.axis_index("c")
    s = lax.axis_index("s")
    cbase = jnp.where(c == 0, s * GC0, NS * GC0 + s * GC1)
    gc = jnp.where(c == 0, GC0, GC1)
    base = s * RPT

    def _zrow(i, carry):
        for j in range(W // 16):
            rows[0, i, pl.ds(j * 16, 16)] = jnp.zeros((16,), jnp.float32)
        return carry

    lax.fori_loop(0, CH, _zrow, 0)

    _stripe_copy(lambda o, n: rows.at[0, pl.ds(0, n)],
                 lambda o, n: acc.at[pl.ds(base + o, n)])
    plsc.subcore_barrier()

    def _issue_idx(g, i):
        pltpu.async_copy(sd2.at[cbase + g, 0], srcb[i], semi[i])
        pltpu.async_copy(sd2.at[cbase + g, 1], dstb[i], semi[i])

    def _wait_idx(g, i):
        pltpu.make_async_copy(sd2.at[cbase + g, 0], srcb[i], semi[i]).wait()
        pltpu.make_async_copy(sd2.at[cbase + g, 1], dstb[i], semi[i]).wait()

    def _issue_gather(i, b):
        pltpu.async_copy(table.at[srcb[i]], rows.at[b], semg[b])

    def _wait_gather(i, b):
        pltpu.make_async_copy(table.at[srcb[i]], rows.at[b], semg[b]).wait()

    def _scat(i, b):
        pltpu.sync_copy(rows.at[b], acc.at[dstb[i]], add=True)

    for g in range(4):
        _issue_idx(g, g)
    _wait_idx(0, 0)
    _issue_gather(0, 0)

    def _body(gg, carry):
        for k in range(4):
            cc = gg * 4 + k
            r = k % 2
            r1 = (k + 1) % 2
            i1 = (k + 1) % 4

            @pl.when(cc + 1 < gc)
            def _():
                _wait_idx(cc + 1, i1)
                _issue_gather(i1, r1)

            _wait_gather(k, r)
            _scat(k, r)

            @pl.when(cc + 4 < gc)
            def _():
                _issue_idx(cc + 4, k)
        return carry

    lax.fori_loop(0, gc // 4, _body, 0)
    plsc.subcore_barrier()

    _stripe_copy(lambda o, n: acc.at[pl.ds(base + o, n)],
                 lambda o, n: psum.at[c, pl.ds(base + o, n)])


def _cnt_body(sd2, pcnt, acc, db0, db1, db2, db3, rows, si0, si1, si2, si3):
    dstb = (db0, db1, db2, db3)
    semi = (si0, si1, si2, si3)
    c = lax.axis_index("c")
    s = lax.axis_index("s")
    cbase = jnp.where(c == 0, s * GC0, NS * GC0 + s * GC1)
    gc = jnp.where(c == 0, GC0, GC1)
    base = s * RPT

    def _zrow(i, carry):
        for j in range(H // 16):
            rows[i, pl.ds(j * 16, 16)] = jnp.zeros((16,), jnp.float32)
        return carry

    lax.fori_loop(0, CH, _zrow, 0)
    _stripe_copy(lambda o, n: rows.at[pl.ds(0, n)],
                 lambda o, n: acc.at[pl.ds(base + o, n)])

    def _orow(i, carry):
        for j in range(H // 16):
            rows[i, pl.ds(j * 16, 16)] = jnp.ones((16,), jnp.float32)
        return carry

    lax.fori_loop(0, CH, _orow, 0)
    plsc.subcore_barrier()

    def _issue_idx(g, i):
        pltpu.async_copy(sd2.at[cbase + g, 1], dstb[i], semi[i])

    def _wait_idx(g, i):
        pltpu.make_async_copy(sd2.at[cbase + g, 1], dstb[i], semi[i]).wait()

    for g in range(4):
        _issue_idx(g, g)

    def _body(gg, carry):
        for k in range(4):
            cc = gg * 4 + k
            _wait_idx(cc, k)
            pltpu.sync_copy(rows, acc.at[dstb[k]], add=True)

            @pl.when(cc + 4 < gc)
            def _():
                _issue_idx(cc + 4, k)
        return carry

    lax.fori_loop(0, gc // 4, _body, 0)
    plsc.subcore_barrier()
    _stripe_copy(lambda o, n: acc.at[pl.ds(base + o, n)],
                 lambda o, n: pcnt.at[c, pl.ds(base + o, n)])


def _sc_mesh():
    return plsc.VectorSubcoreMesh(core_axis_name="c", subcore_axis_name="s",
                                  num_cores=NC, num_subcores=NS)


def _make_agg(W):
    idx = [pltpu.VMEM((CH,), jnp.int32)] * 8
    sems = [pltpu.SemaphoreType.DMA] * 6
    return pl.kernel(
        functools.partial(_agg_body, W=W),
        out_type=jax.ShapeDtypeStruct((NC, NP, W), jnp.float32),
        mesh=_sc_mesh(),
        scratch_types=[pltpu.VMEM_SHARED((NP, W), jnp.float32)] + idx
        + [pltpu.VMEM((2, CH, W), jnp.float32)] + sems,
        name=f"sage_sc_agg{W}",
    )


def _make_cnt():
    return pl.kernel(
        _cnt_body,
        out_type=jax.ShapeDtypeStruct((NC, NP, H), jnp.float32),
        mesh=_sc_mesh(),
        scratch_types=[pltpu.VMEM_SHARED((NP, H), jnp.float32)]
        + [pltpu.VMEM((CH,), jnp.int32)] * 4
        + [pltpu.VMEM((CH, H), jnp.float32)]
        + [pltpu.SemaphoreType.DMA] * 4,
        name="sage_sc_cnt",
    )


RB = 400
GRID = N // RB


def _inv_cnt(c_r):
    cnt = c_r[0, :, 0:1] + c_r[1, :, 0:1]
    return 1.0 / jnp.maximum(cnt, 1.0)


def _tc_body(x_r, p_r, c_r, ws_r, wn_r, b_r, o_r, *, relu):
    neigh = (p_r[0] + p_r[1]) * _inv_cnt(c_r)
    acc = jnp.dot(x_r[...], ws_r[...], preferred_element_type=jnp.float32)
    acc = acc + jnp.dot(neigh, wn_r[...], preferred_element_type=jnp.float32)
    acc = acc + b_r[...]
    o_r[...] = jnp.maximum(acc, 0.0) if relu else acc


def _node_spec(w):
    return pl.BlockSpec((RB, w), lambda i: (i, 0))


def _part_spec(w):
    return pl.BlockSpec((NC, RB, w), lambda i: (0, i, 0))


def _full_spec(a, b):
    return pl.BlockSpec((a, b), lambda i: (0, 0))


def _make_tc(din, dout, relu, name):
    return pl.pallas_call(
        functools.partial(_tc_body, relu=relu),
        grid=(GRID,),
        in_specs=[_node_spec(din), _part_spec(din), _part_spec(din),
                  _full_spec(din, dout), _full_spec(din, dout),
                  _full_spec(1, dout)],
        out_specs=_node_spec(dout),
        out_shape=jax.ShapeDtypeStruct((N, dout), jnp.float32),
        name=name,
    )


_tc1 = _make_tc(D, H, True, "sage_tc1")
_tc2 = _make_tc(H, H, True, "sage_tc2")
_tc3 = _make_tc(H, C, False, "sage_tc3")


def kernel(feats, edge_index, Ws1, Wn1, b1, Ws2, Wn2, b2, Ws3, Wn3, b3):
    pad = EP - E
    src = jnp.concatenate([edge_index[0], jnp.zeros((pad,), jnp.int32)])
    dst = jnp.concatenate([edge_index[1], jnp.full((pad,), N, jnp.int32)])
    sd2 = jnp.stack([src.reshape(NW * G, CH), dst.reshape(NW * G, CH)], axis=1)
    b1r = b1.reshape(1, H)
    b2r = b2.reshape(1, H)
    b3r = b3.reshape(1, C)

    agg128 = _make_agg(H)
    cnt = _make_cnt()(sd2)
    p1 = agg128(feats, sd2)
    h1 = _tc1(feats, p1, cnt, Ws1, Wn1, b1r)
    p2 = agg128(h1, sd2)
    h2 = _tc2(h1, p2, cnt, Ws2, Wn2, b2r)
    p3 = agg128(h2, sd2)
    return _tc3(h2, p3, cnt, Ws3, Wn3, b3r)


# speedup vs baseline: 1.1398x; 1.0066x over previous
import functools

import jax
import jax.numpy as jnp
from jax import lax
from jax.experimental import pallas as pl
from jax.experimental.pallas import tpu as pltpu
from jax.experimental.pallas import tpu_sc as plsc

N = 10000
E = 320000
D = 128
H = 128
C = 64

NC = 2
NS = 16
NW = NC * NS

CH = 128
EW = 10240
EP = NW * EW
G = EW // CH
GC0 = 152
GC1 = 8
NP = 10112
RPT = NP // NS


def _stripe_copy(src_get, dst_get):
    nfull = RPT // CH
    tail = RPT % CH
    for k in range(nfull):
        off = k * CH
        _src = src_get(off, CH)
        _dst = dst_get(off, CH)
        pltpu.sync_copy(_src, _dst)
    pltpu.sync_copy(src_get(nfull * CH, tail), dst_get(nfull * CH, tail))


def _agg_body(table, sd2, psum, acc, *refs, W):
    srcb = refs[0:4]
    dstb = refs[4:8]
    rows = refs[8]
    semi = refs[9:13]
    semg = refs[13:15]
    c = lax.axis_index("c")
    s = lax.axis_index("s")
    cbase = jnp.where(c == 0, s * GC0, NS * GC0 + s * GC1)
    gc = jnp.where(c == 0, GC0, GC1)
    base = s * RPT

    def _zrow(i, carry):
        for j in range(W // 16):
            rows[0, i, pl.ds(j * 16, 16)] = jnp.zeros((16,), jnp.float32)
        return carry

    lax.fori_loop(0, CH, _zrow, 0)

    _stripe_copy(lambda o, n: rows.at[0, pl.ds(0, n)],
                 lambda o, n: acc.at[pl.ds(base + o, n)])
    plsc.subcore_barrier()

    def _issue_idx(g, i):
        pltpu.async_copy(sd2.at[cbase + g, 0], srcb[i], semi[i])
        pltpu.async_copy(sd2.at[cbase + g, 1], dstb[i], semi[i])

    def _wait_idx(g, i):
        pltpu.make_async_copy(sd2.at[cbase + g, 0], srcb[i], semi[i]).wait()
        pltpu.make_async_copy(sd2.at[cbase + g, 1], dstb[i], semi[i]).wait()

    def _issue_gather(i, b):
        pltpu.async_copy(table.at[srcb[i]], rows.at[b], semg[b])

    def _wait_gather(i, b):
        pltpu.make_async_copy(table.at[srcb[i]], rows.at[b], semg[b]).wait()

    def _scat(i, b):
        pltpu.sync_copy(rows.at[b], acc.at[dstb[i]], add=True)

    for g in range(4):
        _issue_idx(g, g)
    _wait_idx(0, 0)
    _issue_gather(0, 0)

    def _body(gg, carry):
        for k in range(4):
            cc = gg * 4 + k
            r = k % 2
            r1 = (k + 1) % 2
            i1 = (k + 1) % 4

            @pl.when(cc + 1 < gc)
            def _():
                _wait_idx(cc + 1, i1)
                _issue_gather(i1, r1)

            _wait_gather(k, r)
            _scat(k, r)

            @pl.when(cc + 4 < gc)
            def _():
                _issue_idx(cc + 4, k)
        return carry

    lax.fori_loop(0, gc // 4, _body, 0)
    plsc.subcore_barrier()

    _stripe_copy(lambda o, n: acc.at[pl.ds(base + o, n)],
                 lambda o, n: psum.at[c, pl.ds(base + o, n)])


def _cnt_body(sd2, pcnt, acc, db0, db1, db2, db3, rows, si0, si1, si2, si3):
    dstb = (db0, db1, db2, db3)
    semi = (si0, si1, si2, si3)
    c = lax.axis_index("c")
    s = lax.axis_index("s")
    cbase = jnp.where(c == 0, s * GC0, NS * GC0 + s * GC1)
    gc = jnp.where(c == 0, GC0, GC1)
    base = s * RPT

    def _zrow(i, carry):
        for j in range(H // 16):
            rows[i, pl.ds(j * 16, 16)] = jnp.zeros((16,), jnp.float32)
        return carry

    lax.fori_loop(0, CH, _zrow, 0)
    _stripe_copy(lambda o, n: rows.at[pl.ds(0, n)],
                 lambda o, n: acc.at[pl.ds(base + o, n)])

    def _orow(i, carry):
        for j in range(H // 16):
            rows[i, pl.ds(j * 16, 16)] = jnp.ones((16,), jnp.float32)
        return carry

    lax.fori_loop(0, CH, _orow, 0)
    plsc.subcore_barrier()

    def _issue_idx(g, i):
        pltpu.async_copy(sd2.at[cbase + g, 1], dstb[i], semi[i])

    def _wait_idx(g, i):
        pltpu.make_async_copy(sd2.at[cbase + g, 1], dstb[i], semi[i]).wait()

    for g in range(4):
        _issue_idx(g, g)

    def _body(gg, carry):
        for k in range(4):
            cc = gg * 4 + k
            _wait_idx(cc, k)
            pltpu.sync_copy(rows, acc.at[dstb[k]], add=True)

            @pl.when(cc + 4 < gc)
            def _():
                _issue_idx(cc + 4, k)
        return carry

    lax.fori_loop(0, gc // 4, _body, 0)
    plsc.subcore_barrier()
    _stripe_copy(lambda o, n: acc.at[pl.ds(base + o, n)],
                 lambda o, n: pcnt.at[c, pl.ds(base + o, n)])


def _sc_mesh():
    return plsc.VectorSubcoreMesh(core_axis_name="c", subcore_axis_name="s",
                                  num_cores=NC, num_subcores=NS)


def _make_agg(W):
    idx = [pltpu.VMEM((CH,), jnp.int32)] * 8
    sems = [pltpu.SemaphoreType.DMA] * 6
    return pl.kernel(
        functools.partial(_agg_body, W=W),
        out_type=jax.ShapeDtypeStruct((NC, NP, W), jnp.float32),
        mesh=_sc_mesh(),
        scratch_types=[pltpu.VMEM_SHARED((NP, W), jnp.float32)] + idx
        + [pltpu.VMEM((2, CH, W), jnp.float32)] + sems,
        name=f"sage_sc_agg{W}",
    )


def _make_cnt():
    return pl.kernel(
        _cnt_body,
        out_type=jax.ShapeDtypeStruct((NC, NP, H), jnp.float32),
        mesh=_sc_mesh(),
        scratch_types=[pltpu.VMEM_SHARED((NP, H), jnp.float32)]
        + [pltpu.VMEM((CH,), jnp.int32)] * 4
        + [pltpu.VMEM((CH, H), jnp.float32)]
        + [pltpu.SemaphoreType.DMA] * 4,
        name="sage_sc_cnt",
    )


RB = 400
GRID = N // RB


def _inv_cnt(c_r):
    cnt = c_r[0, :, 0:1] + c_r[1, :, 0:1]
    return 1.0 / jnp.maximum(cnt, 1.0)


def _tc_body(x_r, p_r, c_r, ws_r, wn_r, b_r, o_r, *, relu):
    neigh = (p_r[0] + p_r[1]) * _inv_cnt(c_r)
    acc = jnp.dot(x_r[...], ws_r[...], preferred_element_type=jnp.float32)
    acc = acc + jnp.dot(neigh, wn_r[...], preferred_element_type=jnp.float32)
    acc = acc + b_r[...]
    o_r[...] = jnp.maximum(acc, 0.0) if relu else acc


def _node_spec(w):
    return pl.BlockSpec((RB, w), lambda i: (i, 0))


def _part_spec(w):
    return pl.BlockSpec((NC, RB, w), lambda i: (0, i, 0))


def _full_spec(a, b):
    return pl.BlockSpec((a, b), lambda i: (0, 0))


def _make_tc(din, dout, relu, name):
    return pl.pallas_call(
        functools.partial(_tc_body, relu=relu),
        grid=(GRID,),
        in_specs=[_node_spec(din), _part_spec(din), _part_spec(din),
                  _full_spec(din, dout), _full_spec(din, dout),
                  _full_spec(1, dout)],
        out_specs=_node_spec(dout),
        out_shape=jax.ShapeDtypeStruct((N, dout), jnp.float32),
        name=name,
    )


_tc1 = _make_tc(D, H, True, "sage_tc1")
_tc2 = _make_tc(H, H, True, "sage_tc2")
_tc3 = _make_tc(H, C, False, "sage_tc3")


def kernel(feats, edge_index, Ws1, Wn1, b1, Ws2, Wn2, b2, Ws3, Wn3, b3):
    pad = EP - E
    src = jnp.concatenate([edge_index[0], jnp.zeros((pad,), jnp.int32)])
    dst = jnp.concatenate([edge_index[1], jnp.full((pad,), N, jnp.int32)])
    sd2 = jnp.stack([src.reshape(NW * G, CH), dst.reshape(NW * G, CH)], axis=1)
    b1r = b1.reshape(1, H)
    b2r = b2.reshape(1, H)
    b3r = b3.reshape(1, C)

    agg128 = _make_agg(H)
    cnt = _make_cnt()(sd2)
    p1 = agg128(feats, sd2)
    h1 = _tc1(feats, p1, cnt, Ws1, Wn1, b1r)
    p2 = agg128(h1, sd2)
    h2 = _tc2(h1, p2, cnt, Ws2, Wn2, b2r)
    p3 = agg128(h2, sd2)
    return _tc3(h2, p3, cnt, Ws3, Wn3, b3r)


# speedup vs baseline: 1.1826x; 1.0376x over previous
import functools

import jax
import jax.numpy as jnp
from jax import lax
from jax.experimental import pallas as pl
from jax.experimental.pallas import tpu as pltpu
from jax.experimental.pallas import tpu_sc as plsc

N = 10000
E = 320000
D = 128
H = 128
C = 64

NC = 2
NS = 16
NW = NC * NS

CH = 128
EW = 10240
EP = NW * EW
G = EW // CH
GC0 = 152
GC1 = 8
NP = 10112
RPT = NP // NS


def _stripe_copy(src_get, dst_get):
    nfull = RPT // CH
    tail = RPT % CH
    for k in range(nfull):
        off = k * CH
        _src = src_get(off, CH)
        _dst = dst_get(off, CH)
        pltpu.sync_copy(_src, _dst)
    pltpu.sync_copy(src_get(nfull * CH, tail), dst_get(nfull * CH, tail))


def _agg_body(table, sd2, psum, acc, *refs, W):
    srcb = refs[0:4]
    dstb = refs[4:8]
    rows = refs[8]
    semi = refs[9:13]
    semg = refs[13:15]
    c = lax.axis_index("c")
    s = lax.axis_index("s")
    cbase = jnp.where(c == 0, s * GC0, NS * GC0 + s * GC1)
    gc = jnp.where(c == 0, GC0, GC1)
    base = s * RPT

    def _zrow(i, carry):
        for j in range(W // 16):
            rows[0, i, pl.ds(j * 16, 16)] = jnp.zeros((16,), jnp.float32)
        return carry

    lax.fori_loop(0, CH, _zrow, 0)

    _stripe_copy(lambda o, n: rows.at[0, pl.ds(0, n)],
                 lambda o, n: acc.at[pl.ds(base + o, n)])
    plsc.subcore_barrier()

    def _issue_idx(g, i):
        pltpu.async_copy(sd2.at[cbase + g, 0], srcb[i], semi[i])
        pltpu.async_copy(sd2.at[cbase + g, 1], dstb[i], semi[i])

    def _wait_idx(g, i):
        pltpu.make_async_copy(sd2.at[cbase + g, 0], srcb[i], semi[i]).wait()
        pltpu.make_async_copy(sd2.at[cbase + g, 1], dstb[i], semi[i]).wait()

    def _issue_gather(i, b):
        pltpu.async_copy(table.at[srcb[i]], rows.at[b], semg[b])

    def _wait_gather(i, b):
        pltpu.make_async_copy(table.at[srcb[i]], rows.at[b], semg[b]).wait()

    def _scat(i, b):
        pltpu.sync_copy(rows.at[b], acc.at[dstb[i]], add=True)

    for g in range(4):
        _issue_idx(g, g)
    _wait_idx(0, 0)
    _issue_gather(0, 0)

    def _body(gg, carry):
        for k in range(4):
            cc = gg * 4 + k
            r = k % 2
            r1 = (k + 1) % 2
            i1 = (k + 1) % 4

            @pl.when(cc + 1 < gc)
            def _():
                _wait_idx(cc + 1, i1)
                _issue_gather(i1, r1)

            _wait_gather(k, r)
            _scat(k, r)

            @pl.when(cc + 4 < gc)
            def _():
                _issue_idx(cc + 4, k)
        return carry

    lax.fori_loop(0, gc // 4, _body, 0)
    plsc.subcore_barrier()

    _stripe_copy(lambda o, n: acc.at[pl.ds(base + o, n)],
                 lambda o, n: psum.at[c, pl.ds(base + o, n)])


def _cnt_body(sd2, pcnt, acc, db0, db1, db2, db3, rows, si0, si1, si2, si3):
    dstb = (db0, db1, db2, db3)
    semi = (si0, si1, si2, si3)
    c = lax.axis_index("c")
    s = lax.axis_index("s")
    cbase = (c * NS + s) * G
    gc = G
    base = s * RPT

    def _zrow(i, carry):
        for j in range(H // 16):
            rows[i, pl.ds(j * 16, 16)] = jnp.zeros((16,), jnp.float32)
        return carry

    lax.fori_loop(0, CH, _zrow, 0)
    _stripe_copy(lambda o, n: rows.at[pl.ds(0, n)],
                 lambda o, n: acc.at[pl.ds(base + o, n)])

    def _orow(i, carry):
        for j in range(H // 16):
            rows[i, pl.ds(j * 16, 16)] = jnp.ones((16,), jnp.float32)
        return carry

    lax.fori_loop(0, CH, _orow, 0)
    plsc.subcore_barrier()

    def _issue_idx(g, i):
        pltpu.async_copy(sd2.at[cbase + g, 1], dstb[i], semi[i])

    def _wait_idx(g, i):
        pltpu.make_async_copy(sd2.at[cbase + g, 1], dstb[i], semi[i]).wait()

    for g in range(4):
        _issue_idx(g, g)

    def _body(gg, carry):
        for k in range(4):
            cc = gg * 4 + k
            _wait_idx(cc, k)
            pltpu.sync_copy(rows, acc.at[dstb[k]], add=True)

            @pl.when(cc + 4 < gc)
            def _():
                _issue_idx(cc + 4, k)
        return carry

    lax.fori_loop(0, gc // 4, _body, 0)
    plsc.subcore_barrier()
    _stripe_copy(lambda o, n: acc.at[pl.ds(base + o, n)],
                 lambda o, n: pcnt.at[c, pl.ds(base + o, n)])


def _sc_mesh():
    return plsc.VectorSubcoreMesh(core_axis_name="c", subcore_axis_name="s",
                                  num_cores=NC, num_subcores=NS)


def _make_agg(W):
    idx = [pltpu.VMEM((CH,), jnp.int32)] * 8
    sems = [pltpu.SemaphoreType.DMA] * 6
    return pl.kernel(
        functools.partial(_agg_body, W=W),
        out_type=jax.ShapeDtypeStruct((NC, NP, W), jnp.float32),
        mesh=_sc_mesh(),
        scratch_types=[pltpu.VMEM_SHARED((NP, W), jnp.float32)] + idx
        + [pltpu.VMEM((2, CH, W), jnp.float32)] + sems,
        name=f"sage_sc_agg{W}",
    )


def _make_cnt():
    return pl.kernel(
        _cnt_body,
        out_type=jax.ShapeDtypeStruct((NC, NP, H), jnp.float32),
        mesh=_sc_mesh(),
        scratch_types=[pltpu.VMEM_SHARED((NP, H), jnp.float32)]
        + [pltpu.VMEM((CH,), jnp.int32)] * 4
        + [pltpu.VMEM((CH, H), jnp.float32)]
        + [pltpu.SemaphoreType.DMA] * 4,
        name="sage_sc_cnt",
    )


RB = 400
GRID = N // RB


def _inv_cnt(c_r):
    cnt = c_r[0, :, 0:1] + c_r[1, :, 0:1]
    return 1.0 / jnp.maximum(cnt, 1.0)


def _tc_body(x_r, p_r, c_r, ws_r, wn_r, b_r, o_r, *, relu):
    neigh = (p_r[0] + p_r[1]) * _inv_cnt(c_r)
    acc = jnp.dot(x_r[...], ws_r[...], preferred_element_type=jnp.float32)
    acc = acc + jnp.dot(neigh, wn_r[...], preferred_element_type=jnp.float32)
    acc = acc + b_r[...]
    o_r[...] = jnp.maximum(acc, 0.0) if relu else acc


def _node_spec(w):
    return pl.BlockSpec((RB, w), lambda i: (i, 0))


def _part_spec(w):
    return pl.BlockSpec((NC, RB, w), lambda i: (0, i, 0))


def _full_spec(a, b):
    return pl.BlockSpec((a, b), lambda i: (0, 0))


def _make_tc(din, dout, relu, name):
    return pl.pallas_call(
        functools.partial(_tc_body, relu=relu),
        grid=(GRID,),
        in_specs=[_node_spec(din), _part_spec(din), _part_spec(din),
                  _full_spec(din, dout), _full_spec(din, dout),
                  _full_spec(1, dout)],
        out_specs=_node_spec(dout),
        out_shape=jax.ShapeDtypeStruct((N, dout), jnp.float32),
        name=name,
    )


_tc1 = _make_tc(D, H, True, "sage_tc1")
_tc2 = _make_tc(H, H, True, "sage_tc2")
_tc3 = _make_tc(H, C, False, "sage_tc3")


def kernel(feats, edge_index, Ws1, Wn1, b1, Ws2, Wn2, b2, Ws3, Wn3, b3):
    pad = EP - E
    src = jnp.concatenate([edge_index[0], jnp.zeros((pad,), jnp.int32)])
    dst = jnp.concatenate([edge_index[1], jnp.full((pad,), N, jnp.int32)])
    sd2 = jnp.stack([src.reshape(NW * G, CH), dst.reshape(NW * G, CH)], axis=1)
    b1r = b1.reshape(1, H)
    b2r = b2.reshape(1, H)
    b3r = b3.reshape(1, C)

    agg128 = _make_agg(H)
    cnt = _make_cnt()(sd2)
    p1 = agg128(feats, sd2)
    h1 = _tc1(feats, p1, cnt, Ws1, Wn1, b1r)
    p2 = agg128(h1, sd2)
    h2 = _tc2(h1, p2, cnt, Ws2, Wn2, b2r)
    p3 = agg128(h2, sd2)
    return _tc3(h2, p3, cnt, Ws3, Wn3, b3r)
